# baseline (speedup 1.0000x reference)
import jax
import jax.numpy as jnp
from jax.experimental import pallas as pl
from jax.experimental.pallas import tpu as pltpu

B, N, DIM, E = 4, 4096, 2048, 8
CAP = 512
TOK_BLK = 1024
N_TOK = B * N
SUB = 8
LANE = N // SUB


def _router_kernel(x_ref, w_ref, b_ref, pt_ref):
    logits = jax.lax.dot_general(
        w_ref[...], x_ref[...],
        dimension_numbers=(((1,), (1,)), ((), ())),
        preferred_element_type=jnp.float32,
    )
    logits = logits + b_ref[...].reshape(E, 1)
    m = jnp.max(logits, axis=0, keepdims=True)
    e = jnp.exp(logits - m)
    pt_ref[...] = e / jnp.sum(e, axis=0, keepdims=True)


def _count_ge(keys, thr):
    return jnp.sum((keys >= thr).astype(jnp.float32), axis=(1, 2),
                   keepdims=True)


def _route_kernel(pt_ref, mask_ref, ep_ref, sun_ref, mc_ref):
    idx = (LANE * jax.lax.broadcasted_iota(jnp.int32, (1, SUB, LANE), 1)
           + jax.lax.broadcasted_iota(jnp.int32, (1, SUB, LANE), 2))
    unassigned = jnp.ones((1, SUB, LANE), dtype=jnp.int32)
    maskv = jnp.full((1, SUB, LANE), -1, dtype=jnp.int32)

    for j in reversed(range(E)):
        p_j = pt_ref[j]

        def _search(unassigned=unassigned, p_j=p_j):
            keys = jnp.where(unassigned > 0,
                             jax.lax.bitcast_convert_type(p_j, jnp.int32) + 1,
                             jnp.int32(0))

            kth = jnp.zeros((B, 1, 1), dtype=jnp.int32)
            for r in range(15):
                sh = 28 - 2 * r
                cnts = [_count_ge(keys, kth + (d << sh)) for d in range(1, 4)]
                dwin = sum((c >= float(CAP)).astype(jnp.int32) for c in cnts)
                kth = kth + (dwin << sh)

            cg = _count_ge(keys, kth + 1)
            need = float(CAP) - cg
            ties = keys == kth

            @pl.when(jnp.max(need) <= 1.0)
            def _():
                mc_ref[...] = jnp.min(jnp.where(ties, idx, jnp.int32(N)),
                                      axis=(1, 2), keepdims=True)

            @pl.when(jnp.max(need) > 1.0)
            def _():
                def f_lt(m):
                    return jnp.sum((ties & (idx < m)).astype(jnp.float32),
                                   axis=(1, 2), keepdims=True)

                mcut = jnp.zeros((B, 1, 1), dtype=jnp.int32)
                for r in range(6):
                    sh = 10 - 2 * r
                    fs = [f_lt(mcut + (d << sh)) for d in range(1, 4)]
                    dwin = sum((f < need).astype(jnp.int32) for f in fs)
                    mcut = mcut + (dwin << sh)
                mc_ref[...] = mcut

            sel = (keys > kth) | (ties & (idx <= mc_ref[...]))
            return jnp.max(sel.astype(jnp.int32), axis=0, keepdims=True)

        any_unassigned = jnp.sum(unassigned) > 0

        @pl.when(any_unassigned)
        def _():
            sun_ref[...] = _search()

        @pl.when(jnp.logical_not(any_unassigned))
        def _():
            sun_ref[...] = (idx < CAP).astype(jnp.int32)

        s_union = sun_ref[...]
        maskv = jnp.where(s_union > 0, jnp.int32(j), maskv)
        unassigned = unassigned * (1 - s_union)

    maskv = jnp.where(maskv == -1, 0, maskv)
    mask_ref[...] = jnp.broadcast_to(maskv, (B, SUB, LANE))
    ep = jnp.zeros((B, SUB, LANE), dtype=jnp.float32)
    for j in range(E):
        ep = ep + jnp.where(maskv == j, pt_ref[j], 0.0)
    ep_ref[...] = ep


@jax.jit
def kernel(input_tokens, W, b):
    x = input_tokens.reshape(N_TOK, DIM)
    b2 = b.reshape(1, E)
    probs_t = pl.pallas_call(
        _router_kernel,
        grid=(N_TOK // TOK_BLK,),
        in_specs=[
            pl.BlockSpec((TOK_BLK, DIM), lambda i: (i, 0)),
            pl.BlockSpec((E, DIM), lambda i: (0, 0)),
            pl.BlockSpec((1, E), lambda i: (0, 0)),
        ],
        out_specs=pl.BlockSpec((E, TOK_BLK), lambda i: (0, i)),
        out_shape=jax.ShapeDtypeStruct((E, N_TOK), jnp.float32),
    )(x, W, b2)
    probs_t = probs_t.reshape(E, B, SUB, LANE)
    mask, ep = pl.pallas_call(
        _route_kernel,
        out_shape=(
            jax.ShapeDtypeStruct((B, SUB, LANE), jnp.int32),
            jax.ShapeDtypeStruct((B, SUB, LANE), jnp.float32),
        ),
        scratch_shapes=[pltpu.VMEM((1, SUB, LANE), jnp.int32),
                        pltpu.VMEM((B, 1, 1), jnp.int32)],
    )(probs_t)
    return mask.reshape(B, N), ep.reshape(B, N)


# speedup vs baseline: 1.0928x; 1.0928x over previous
import jax
import jax.numpy as jnp
from jax.experimental import pallas as pl
from jax.experimental.pallas import tpu as pltpu

B, N, DIM, E = 4, 4096, 2048, 8
CAP = 512
TOK_BLK = 1024
N_TOK = B * N
SUB = 8
LANE = N // SUB
GRID = N_TOK // TOK_BLK
BLK_PER_ROW = N // TOK_BLK


def _count_ge(keys, thr):
    return jnp.sum((keys >= thr).astype(jnp.float32), axis=(1, 2),
                   keepdims=True)


def _routing_epilogue(pts_ref, mask_ref, ep_ref, sun_ref, mc_ref):
    pts = [None] * E
    for j in range(E):
        rows = []
        for b in range(B):
            rows.append(jnp.concatenate(
                [pts_ref[BLK_PER_ROW * b + m, j].reshape(2, LANE)
                 for m in range(BLK_PER_ROW)], axis=0))
        pts[j] = jnp.stack(rows, axis=0)

    idx = (LANE * jax.lax.broadcasted_iota(jnp.int32, (1, SUB, LANE), 1)
           + jax.lax.broadcasted_iota(jnp.int32, (1, SUB, LANE), 2))
    unassigned = jnp.ones((1, SUB, LANE), dtype=jnp.int32)
    maskv = jnp.full((1, SUB, LANE), -1, dtype=jnp.int32)

    for j in reversed(range(E)):
        p_j = pts[j]

        def _search(unassigned=unassigned, p_j=p_j):
            keys = jnp.where(unassigned > 0,
                             jax.lax.bitcast_convert_type(p_j, jnp.int32) + 1,
                             jnp.int32(0))

            kth = jnp.zeros((B, 1, 1), dtype=jnp.int32)
            for r in range(10):
                sh = 27 - 3 * r
                cnts = [_count_ge(keys, kth + (d << sh)) for d in range(1, 8)]
                dwin = sum((c >= float(CAP)).astype(jnp.int32) for c in cnts)
                kth = kth + (dwin << sh)

            cg = _count_ge(keys, kth + 1)
            need = float(CAP) - cg
            ties = keys == kth

            @pl.when(jnp.max(need) <= 1.0)
            def _():
                mc_ref[...] = jnp.min(jnp.where(ties, idx, jnp.int32(N)),
                                      axis=(1, 2), keepdims=True)

            @pl.when(jnp.max(need) > 1.0)
            def _():
                def f_lt(m):
                    return jnp.sum((ties & (idx < m)).astype(jnp.float32),
                                   axis=(1, 2), keepdims=True)

                mcut = jnp.zeros((B, 1, 1), dtype=jnp.int32)
                for r in range(4):
                    sh = 9 - 3 * r
                    fs = [f_lt(mcut + (d << sh)) for d in range(1, 8)]
                    dwin = sum((f < need).astype(jnp.int32) for f in fs)
                    mcut = mcut + (dwin << sh)
                mc_ref[...] = mcut

            sel = (keys > kth) | (ties & (idx <= mc_ref[...]))
            return jnp.max(sel.astype(jnp.int32), axis=0, keepdims=True)

        any_unassigned = jnp.sum(unassigned) > 0

        @pl.when(any_unassigned)
        def _():
            sun_ref[...] = _search()

        @pl.when(jnp.logical_not(any_unassigned))
        def _():
            sun_ref[...] = (idx < CAP).astype(jnp.int32)

        s_union = sun_ref[...]
        maskv = jnp.where(s_union > 0, jnp.int32(j), maskv)
        unassigned = unassigned * (1 - s_union)

    maskv = jnp.where(maskv == -1, 0, maskv)
    mask_ref[...] = jnp.broadcast_to(maskv, (B, SUB, LANE))
    ep = jnp.zeros((B, SUB, LANE), dtype=jnp.float32)
    for j in range(E):
        ep = ep + jnp.where(maskv == j, pts[j], 0.0)
    ep_ref[...] = ep


def _fused_kernel(x_ref, w_ref, b_ref, mask_ref, ep_ref,
                  pts_ref, sun_ref, mc_ref):
    i = pl.program_id(0)
    logits = jax.lax.dot_general(
        w_ref[...], x_ref[...],
        dimension_numbers=(((1,), (1,)), ((), ())),
        preferred_element_type=jnp.float32,
    )
    logits = logits + b_ref[...].reshape(E, 1)
    m = jnp.max(logits, axis=0, keepdims=True)
    e = jnp.exp(logits - m)
    pts_ref[i] = e / jnp.sum(e, axis=0, keepdims=True)

    @pl.when(i == GRID - 1)
    def _():
        _routing_epilogue(pts_ref, mask_ref, ep_ref, sun_ref, mc_ref)


@jax.jit
def kernel(input_tokens, W, b):
    x = input_tokens.reshape(N_TOK, DIM)
    b2 = b.reshape(1, E)
    mask, ep = pl.pallas_call(
        _fused_kernel,
        grid=(GRID,),
        in_specs=[
            pl.BlockSpec((TOK_BLK, DIM), lambda i: (i, 0)),
            pl.BlockSpec((E, DIM), lambda i: (0, 0)),
            pl.BlockSpec((1, E), lambda i: (0, 0)),
        ],
        out_specs=(
            pl.BlockSpec((B, SUB, LANE), lambda i: (0, 0, 0)),
            pl.BlockSpec((B, SUB, LANE), lambda i: (0, 0, 0)),
        ),
        out_shape=(
            jax.ShapeDtypeStruct((B, SUB, LANE), jnp.int32),
            jax.ShapeDtypeStruct((B, SUB, LANE), jnp.float32),
        ),
        scratch_shapes=[
            pltpu.VMEM((GRID, E, TOK_BLK), jnp.float32),
            pltpu.VMEM((1, SUB, LANE), jnp.int32),
            pltpu.VMEM((B, 1, 1), jnp.int32),
        ],
    )(x, W, b2)
    return mask.reshape(B, N), ep.reshape(B, N)


# speedup vs baseline: 1.1172x; 1.0223x over previous
import jax
import jax.numpy as jnp
from jax.experimental import pallas as pl
from jax.experimental.pallas import tpu as pltpu

B, N, DIM, E = 4, 4096, 2048, 8
CAP = 512
TOK_BLK = 1024
N_TOK = B * N
SUB = 8
LANE = N // SUB
GRID = N_TOK // TOK_BLK
BLK_PER_ROW = N // TOK_BLK


def _count_ge(keys, thr):
    return jnp.sum((keys >= thr).astype(jnp.float32), axis=(1, 2),
                   keepdims=True)


def _routing_epilogue(pts_ref, mask_ref, ep_ref, sun_ref, mc_ref, kv_ref):
    pts = [None] * E
    for j in range(E):
        rows = []
        for b in range(B):
            rows.append(jnp.concatenate(
                [pts_ref[BLK_PER_ROW * b + m, j].reshape(2, LANE)
                 for m in range(BLK_PER_ROW)], axis=0))
        pts[j] = jnp.stack(rows, axis=0)

    idx = (LANE * jax.lax.broadcasted_iota(jnp.int32, (1, SUB, LANE), 1)
           + jax.lax.broadcasted_iota(jnp.int32, (1, SUB, LANE), 2))
    unassigned = jnp.ones((1, SUB, LANE), dtype=jnp.int32)
    maskv = jnp.full((1, SUB, LANE), -1, dtype=jnp.int32)

    for j in reversed(range(E)):
        p_j = pts[j]

        def _search(unassigned=unassigned, p_j=p_j, n_un=None):
            keys = jnp.where(unassigned > 0,
                             jax.lax.bitcast_convert_type(p_j, jnp.int32) + 1,
                             jnp.int32(0))

            @pl.when(n_un >= CAP)
            def _():
                kth = jnp.zeros((B, 1, 1), dtype=jnp.int32)
                for r in range(10):
                    sh = 27 - 3 * r
                    cnts = [_count_ge(keys, kth + (d << sh))
                            for d in range(1, 8)]
                    dwin = sum((c >= float(CAP)).astype(jnp.int32)
                               for c in cnts)
                    kth = kth + (dwin << sh)
                kv_ref[...] = kth

            @pl.when(n_un < CAP)
            def _():
                kv_ref[...] = jnp.zeros((B, 1, 1), dtype=jnp.int32)

            kth = kv_ref[...]
            cg = _count_ge(keys, kth + 1)
            need = float(CAP) - cg
            ties = keys == kth

            @pl.when(jnp.max(need) <= 1.0)
            def _():
                mc_ref[...] = jnp.min(jnp.where(ties, idx, jnp.int32(N)),
                                      axis=(1, 2), keepdims=True)

            @pl.when(jnp.max(need) > 1.0)
            def _():
                def f_lt(m):
                    return jnp.sum((ties & (idx < m)).astype(jnp.float32),
                                   axis=(1, 2), keepdims=True)

                mcut = jnp.zeros((B, 1, 1), dtype=jnp.int32)
                for r in range(4):
                    sh = 9 - 3 * r
                    fs = [f_lt(mcut + (d << sh)) for d in range(1, 8)]
                    dwin = sum((f < need).astype(jnp.int32) for f in fs)
                    mcut = mcut + (dwin << sh)
                mc_ref[...] = mcut

            sel = (keys > kth) | (ties & (idx <= mc_ref[...]))
            return jnp.max(sel.astype(jnp.int32), axis=0, keepdims=True)

        n_un = jnp.sum(unassigned)

        @pl.when(n_un > 0)
        def _():
            sun_ref[...] = _search(n_un=n_un)

        @pl.when(n_un == 0)
        def _():
            sun_ref[...] = (idx < CAP).astype(jnp.int32)

        s_union = sun_ref[...]
        maskv = jnp.where(s_union > 0, jnp.int32(j), maskv)
        unassigned = unassigned * (1 - s_union)

    maskv = jnp.where(maskv == -1, 0, maskv)
    mask_ref[...] = jnp.broadcast_to(maskv, (B, SUB, LANE))
    ep = jnp.zeros((B, SUB, LANE), dtype=jnp.float32)
    for j in range(E):
        ep = ep + jnp.where(maskv == j, pts[j], 0.0)
    ep_ref[...] = ep


def _fused_kernel(x_ref, w_ref, b_ref, mask_ref, ep_ref,
                  pts_ref, sun_ref, mc_ref, kv_ref):
    i = pl.program_id(0)
    logits = jax.lax.dot_general(
        w_ref[...], x_ref[...],
        dimension_numbers=(((1,), (1,)), ((), ())),
        preferred_element_type=jnp.float32,
    )
    logits = logits + b_ref[...].reshape(E, 1)
    m = jnp.max(logits, axis=0, keepdims=True)
    e = jnp.exp(logits - m)
    pts_ref[i] = e / jnp.sum(e, axis=0, keepdims=True)

    @pl.when(i == GRID - 1)
    def _():
        _routing_epilogue(pts_ref, mask_ref, ep_ref, sun_ref, mc_ref, kv_ref)


@jax.jit
def kernel(input_tokens, W, b):
    x = input_tokens.reshape(N_TOK, DIM)
    b2 = b.reshape(1, E)
    mask, ep = pl.pallas_call(
        _fused_kernel,
        grid=(GRID,),
        in_specs=[
            pl.BlockSpec((TOK_BLK, DIM), lambda i: (i, 0)),
            pl.BlockSpec((E, DIM), lambda i: (0, 0)),
            pl.BlockSpec((1, E), lambda i: (0, 0)),
        ],
        out_specs=(
            pl.BlockSpec((B, SUB, LANE), lambda i: (0, 0, 0)),
            pl.BlockSpec((B, SUB, LANE), lambda i: (0, 0, 0)),
        ),
        out_shape=(
            jax.ShapeDtypeStruct((B, SUB, LANE), jnp.int32),
            jax.ShapeDtypeStruct((B, SUB, LANE), jnp.float32),
        ),
        scratch_shapes=[
            pltpu.VMEM((GRID, E, TOK_BLK), jnp.float32),
            pltpu.VMEM((1, SUB, LANE), jnp.int32),
            pltpu.VMEM((B, 1, 1), jnp.int32),
            pltpu.VMEM((B, 1, 1), jnp.int32),
        ],
    )(x, W, b2)
    return mask.reshape(B, N), ep.reshape(B, N)
